# baseline (speedup 1.0000x reference)
import functools

import jax
import jax.numpy as jnp
from jax import lax
from jax.experimental import pallas as pl
from jax.experimental.pallas import tpu as pltpu
from jax.experimental.pallas import tpu_sc as plsc

D = 64
HW = 256
K = 3
BB = 256
IDX_MINOR = 128
OUT_PAD = 257
NW = 32


def _sc_bow_embed(idx, table, batch):
    n_units = HW * (batch // BB) // NW
    chunks = batch // BB

    mesh = plsc.VectorSubcoreMesh(core_axis_name="c", subcore_axis_name="s")

    @functools.partial(
        pl.kernel,
        out_type=jax.ShapeDtypeStruct((D, 16, 2, batch // 128, 8, 128),
                                      jnp.float32),
        mesh=mesh,
        compiler_params=pltpu.CompilerParams(
            needs_layout_passes=False, use_tc_tiling_on_sc=False),
        scratch_types=[
            pltpu.VMEM((2, K, 2, IDX_MINOR), jnp.int32),
            pltpu.VMEM((2, K * BB, D), jnp.float32),
            pltpu.VMEM((D, OUT_PAD), jnp.float32),
            pltpu.SemaphoreType.DMA,
            pltpu.SemaphoreType.DMA,
            pltpu.SemaphoreType.DMA,
        ],
    )
    def body(idx_hbm, table_hbm, out_hbm, idx_v, rows_v, out_t, g0, g1, osem):
        wid = lax.axis_index("s") * 2 + lax.axis_index("c")
        lane = lax.iota(jnp.int32, 16)
        third = jnp.float32(1.0 / 3.0)
        e_rows = [c * 16 + lane for c in range(4)]
        gsem = (g0, g1)

        def unit_hwb(u):
            hw = wid * (n_units // chunks) + (u // chunks)
            cb = u % chunks
            return hw // 16, hw % 16, cb

        def idx_copy(u, buf):
            h, w, cb = unit_hwb(u)
            pltpu.sync_copy(
                idx_hbm.at[:, h, w, pl.ds(cb * 2, 2)], idx_v.at[buf])

        def fire(buf):
            for k in range(K):
                for j in range(2):
                    pltpu.async_copy(
                        table_hbm.at[idx_v.at[buf, k, j]],
                        rows_v.at[buf, pl.ds(k * BB + j * IDX_MINOR,
                                             IDX_MINOR)],
                        gsem[buf],
                    )

        def drain(buf):
            for k in range(K):
                for j in range(2):
                    pltpu.make_async_copy(
                        table_hbm.at[idx_v.at[buf, k, j]],
                        rows_v.at[buf, pl.ds(k * BB + j * IDX_MINOR,
                                             IDX_MINOR)],
                        gsem[buf],
                    ).wait()

        def out_dmas(u):
            h, w, cb = unit_hwb(u)
            wt, w8 = w // 8, w % 8
            return [
                pltpu.make_async_copy(
                    out_t.at[:, pl.ds(j * IDX_MINOR, IDX_MINOR)],
                    out_hbm.at[:, h, wt, 2 * cb + j, w8],
                    osem,
                )
                for j in range(2)
            ]

        def compute(buf):
            rv = rows_v.at[buf]

            @plsc.parallel_loop(0, BB, 1, unroll=8)
            def _(p):
                col = jnp.zeros((16,), jnp.int32) + p
                for c in range(4):
                    sl = pl.ds(c * 16, 16)
                    v = (rv[p, sl] + rv[BB + p, sl]
                         + rv[2 * BB + p, sl]) * third
                    plsc.store_scatter(out_t, [e_rows[c], col], v)

        idx_copy(0, 0)
        fire(0)

        def pair_body(pr, _):
            i = pr * 2
            for par in (0, 1):
                u = i + par
                nxt = u + 1

                @pl.when(nxt < n_units)
                def _():
                    idx_copy(nxt, 1 - par)
                    fire(1 - par)

                drain(par)

                @pl.when(u > 0)
                def _():
                    for cp in out_dmas(u - 1):
                        cp.wait()

                compute(par)
                for cp in out_dmas(u):
                    cp.start()
            return 0

        lax.fori_loop(0, n_units // 2, pair_body, 0)
        for cp in out_dmas(n_units - 1):
            cp.wait()

    return body(idx, table)


def kernel(inputs, table):
    b, k, h, w = inputs.shape
    idx = inputs.transpose(1, 2, 3, 0).reshape(k, h, w, b // 128, 128)
    out = _sc_bow_embed(idx, table, b)
    out = out.transpose(3, 5, 0, 1, 2, 4)
    return out.reshape(b, D, h, w)


# speedup vs baseline: 1.0772x; 1.0772x over previous
import functools

import jax
import jax.numpy as jnp
from jax import lax
from jax.experimental import pallas as pl
from jax.experimental.pallas import tpu as pltpu
from jax.experimental.pallas import tpu_sc as plsc

D = 64
HW = 256
K = 3
BB = 256
IDX_MINOR = 128
OUT_PAD = 257
NW = 32


def _sc_bow_embed(idx, table, batch):
    n_units = HW * (batch // BB) // NW
    chunks = batch // BB

    mesh = plsc.VectorSubcoreMesh(core_axis_name="c", subcore_axis_name="s")

    @functools.partial(
        pl.kernel,
        out_type=jax.ShapeDtypeStruct((D, 16, 2, batch // 128, 8, 128),
                                      jnp.float32),
        mesh=mesh,
        compiler_params=pltpu.CompilerParams(
            needs_layout_passes=False, use_tc_tiling_on_sc=False),
        scratch_types=[
            pltpu.VMEM((4, K, 2, IDX_MINOR), jnp.int32),
            pltpu.VMEM((2, K * BB, D), jnp.float32),
            pltpu.VMEM((D, OUT_PAD), jnp.float32),
            pltpu.SemaphoreType.DMA,
            pltpu.SemaphoreType.DMA,
            pltpu.SemaphoreType.DMA,
            pltpu.SemaphoreType.DMA,
        ],
    )
    def body(idx_hbm, table_hbm, out_hbm, idx_v, rows_v, out_t,
             g0, g1, osem, isem):
        wid = lax.axis_index("s") * 2 + lax.axis_index("c")
        lane = lax.iota(jnp.int32, 16)
        third = jnp.float32(1.0 / 3.0)
        e_rows = [c * 16 + lane for c in range(4)]
        gsem = (g0, g1)

        def unit_hwb(u):
            hw = wid * (n_units // chunks) + (u // chunks)
            cb = u % chunks
            return hw // 16, hw % 16, cb

        def idx_dma(u, buf):
            h, w, cb = unit_hwb(u)
            return pltpu.make_async_copy(
                idx_hbm.at[:, h, w, pl.ds(cb * 2, 2)], idx_v.at[buf], isem)

        def fire(rbuf, ibuf):
            for k in range(K):
                for j in range(2):
                    pltpu.async_copy(
                        table_hbm.at[idx_v.at[ibuf, k, j]],
                        rows_v.at[rbuf, pl.ds(k * BB + j * IDX_MINOR,
                                              IDX_MINOR)],
                        gsem[rbuf],
                    )

        def drain(rbuf, ibuf):
            for k in range(K):
                for j in range(2):
                    pltpu.make_async_copy(
                        table_hbm.at[idx_v.at[ibuf, k, j]],
                        rows_v.at[rbuf, pl.ds(k * BB + j * IDX_MINOR,
                                              IDX_MINOR)],
                        gsem[rbuf],
                    ).wait()

        def out_dmas(u):
            h, w, cb = unit_hwb(u)
            wt, w8 = w // 8, w % 8
            return [
                pltpu.make_async_copy(
                    out_t.at[:, pl.ds(j * IDX_MINOR, IDX_MINOR)],
                    out_hbm.at[:, h, wt, 2 * cb + j, w8],
                    osem,
                )
                for j in range(2)
            ]

        def compute(buf):
            rv = rows_v.at[buf]

            @plsc.parallel_loop(0, BB, 1, unroll=4)
            def _(p):
                col = jnp.zeros((16,), jnp.int32) + p
                for c in range(4):
                    sl = pl.ds(c * 16, 16)
                    v = (rv[p, sl] + rv[BB + p, sl]
                         + rv[2 * BB + p, sl]) * third
                    plsc.store_scatter(out_t, [e_rows[c], col], v)

        idx_dma(0, 0).start()
        idx_dma(0, 0).wait()
        fire(0, 0)
        idx_dma(1, 1).start()

        def quad_body(qr, _):
            base = qr * 4
            for par in (0, 1, 2, 3):
                u = base + par
                rbuf = par & 1

                @pl.when(u + 1 < n_units)
                def _():
                    idx_dma(u + 1, (par + 1) & 3).wait()
                    fire(1 - rbuf, (par + 1) & 3)

                @pl.when(u + 2 < n_units)
                def _():
                    idx_dma(u + 2, (par + 2) & 3).start()

                drain(rbuf, par & 3)

                @pl.when(u > 0)
                def _():
                    for cp in out_dmas(u - 1):
                        cp.wait()

                compute(rbuf)
                for cp in out_dmas(u):
                    cp.start()
            return 0

        lax.fori_loop(0, n_units // 4, quad_body, 0)
        for cp in out_dmas(n_units - 1):
            cp.wait()

    return body(idx, table)


def kernel(inputs, table):
    b, k, h, w = inputs.shape
    idx = inputs.transpose(1, 2, 3, 0).reshape(k, h, w, b // 128, 128)
    out = _sc_bow_embed(idx, table, b)
    out = out.transpose(3, 5, 0, 1, 2, 4)
    return out.reshape(b, D, h, w)


# speedup vs baseline: 1.0951x; 1.0166x over previous
import functools

import jax
import jax.numpy as jnp
from jax import lax
from jax.experimental import pallas as pl
from jax.experimental.pallas import tpu as pltpu
from jax.experimental.pallas import tpu_sc as plsc

D = 64
HW = 256
K = 3
BB = 128
OUT_PAD = 129
NW = 32


def _sc_bow_embed(idx, table, batch):
    chunks = batch // BB
    n_units = HW * chunks // NW
    hw_per_w = n_units // chunks

    mesh = plsc.VectorSubcoreMesh(core_axis_name="c", subcore_axis_name="s")

    @functools.partial(
        pl.kernel,
        out_type=jax.ShapeDtypeStruct((D, 16, 2, batch // 128, 8, 128),
                                      jnp.float32),
        mesh=mesh,
        compiler_params=pltpu.CompilerParams(
            needs_layout_passes=False, use_tc_tiling_on_sc=False),
        scratch_types=[
            pltpu.VMEM((4, K, BB), jnp.int32),
            pltpu.VMEM((4, K * BB, D), jnp.float32),
            pltpu.VMEM((D, OUT_PAD), jnp.float32),
            [pltpu.SemaphoreType.DMA] * 4,
            [pltpu.SemaphoreType.DMA] * 4,
            pltpu.SemaphoreType.DMA,
        ],
    )
    def body(idx_hbm, table_hbm, out_hbm, idx_v, rows_v, out_t,
             gsem, isem, osem):
        wid = lax.axis_index("s") * 2 + lax.axis_index("c")
        lane = lax.iota(jnp.int32, 16)
        third = jnp.float32(1.0 / 3.0)
        e_rows = [c * 16 + lane for c in range(4)]

        def unit_hwb(u):
            hw = wid * hw_per_w + (u // chunks)
            cb = u % chunks
            return hw // 16, hw % 16, cb

        def idx_dma(u, buf):
            h, w, cb = unit_hwb(u)
            return pltpu.make_async_copy(
                idx_hbm.at[:, h, w, cb], idx_v.at[buf], isem[buf])

        def fire(buf):
            for k in range(K):
                pltpu.async_copy(
                    table_hbm.at[idx_v.at[buf, k]],
                    rows_v.at[buf, pl.ds(k * BB, BB)],
                    gsem[buf],
                )

        def drain(buf):
            for k in range(K):
                pltpu.make_async_copy(
                    table_hbm.at[idx_v.at[buf, k]],
                    rows_v.at[buf, pl.ds(k * BB, BB)],
                    gsem[buf],
                ).wait()

        def out_dma(u):
            h, w, cb = unit_hwb(u)
            wt, w8 = w // 8, w % 8
            return pltpu.make_async_copy(
                out_t.at[:, pl.ds(0, BB)],
                out_hbm.at[:, h, wt, cb, w8],
                osem,
            )

        def compute(buf):
            rv = rows_v.at[buf]

            @plsc.parallel_loop(0, BB, 1, unroll=4)
            def _(p):
                col = jnp.zeros((16,), jnp.int32) + p
                for c in range(4):
                    sl = pl.ds(c * 16, 16)
                    v = (rv[p, sl] + rv[BB + p, sl]
                         + rv[2 * BB + p, sl]) * third
                    plsc.store_scatter(out_t, [e_rows[c], col], v)

        idx_dma(0, 0).start()
        idx_dma(0, 0).wait()
        fire(0)
        idx_dma(1, 1).start()
        idx_dma(1, 1).wait()
        fire(1)
        idx_dma(2, 2).start()

        def quad_body(qr, _):
            base = qr * 4
            for par in (0, 1, 2, 3):
                u = base + par

                @pl.when(u + 2 < n_units)
                def _():
                    idx_dma(u + 2, (par + 2) & 3).wait()
                    fire((par + 2) & 3)

                @pl.when(u + 3 < n_units)
                def _():
                    idx_dma(u + 3, (par + 3) & 3).start()

                drain(par)

                @pl.when(u > 0)
                def _():
                    out_dma(u - 1).wait()

                compute(par)
                out_dma(u).start()
            return 0

        lax.fori_loop(0, n_units // 4, quad_body, 0)
        out_dma(n_units - 1).wait()

    return body(idx, table)


def kernel(inputs, table):
    b, k, h, w = inputs.shape
    idx = inputs.transpose(1, 2, 3, 0).reshape(k, h, w, b // 128, 128)
    out = _sc_bow_embed(idx, table, b)
    out = out.transpose(3, 5, 0, 1, 2, 4)
    return out.reshape(b, D, h, w)
